# baseline (speedup 1.0000x reference)
import functools

import jax
import jax.numpy as jnp
from jax import lax
from jax.experimental import pallas as pl
from jax.experimental.pallas import tpu as pltpu
from jax.experimental.pallas import tpu_sc as plsc

B = 16384
D = 64
NC = 2
NS = 16
NW = NC * NS
BPW = B // NW
CH = 64
NCHUNK = BPW // CH
LANES = 16
UNROLL = 4


def _cs_body(rel_ref, cos_ref, sin_ref):
    cos_ref[...] = jnp.cos(rel_ref[...])
    sin_ref[...] = jnp.sin(rel_ref[...])


def _cos_sin_tables(relation):
    return pl.pallas_call(
        _cs_body,
        out_shape=(
            jax.ShapeDtypeStruct(relation.shape, jnp.float32),
            jax.ShapeDtypeStruct(relation.shape, jnp.float32),
        ),
    )(relation)


_mesh = plsc.VectorSubcoreMesh(core_axis_name="c", subcore_axis_name="s")


@functools.partial(
    pl.kernel,
    mesh=_mesh,
    compiler_params=pltpu.CompilerParams(needs_layout_passes=False),
    out_type=(
        jax.ShapeDtypeStruct((D, B), jnp.float32),
        jax.ShapeDtypeStruct((D, B), jnp.float32),
    ),
    scratch_types=[
        pltpu.VMEM((BPW,), jnp.int32),
        pltpu.VMEM((BPW,), jnp.int32),
        pltpu.VMEM((BPW,), jnp.int32),
        pltpu.VMEM((BPW,), jnp.int32),
        pltpu.VMEM((CH, 4 * D), jnp.float32),
        pltpu.VMEM((CH, 4 * D), jnp.float32),
        pltpu.VMEM((CH, 2 * D), jnp.float32),
        pltpu.VMEM((CH, 2 * D), jnp.float32),
        pltpu.VMEM((D, BPW), jnp.float32),
        pltpu.VMEM((D, BPW), jnp.float32),
        pltpu.SemaphoreType.DMA,
    ],
)
def _rotate_sc(e1_hbm, r_hbm, er4_hbm, ei4_hbm, cos2_hbm, sin2_hbm,
               outr_hbm, outi_hbm,
               idx1_v, idx2_v, p1_v, p2_v, erq_v, eiq_v, cosp_v, sinp_v,
               or_v, oi_v, sem):
    wid = lax.axis_index("s") * NC + lax.axis_index("c")
    base = wid * BPW

    pltpu.sync_copy(e1_hbm.at[pl.ds(base, BPW)], idx1_v)
    pltpu.sync_copy(r_hbm.at[pl.ds(base, BPW)], idx2_v)

    for v in range(BPW // LANES):
        sl = pl.ds(v * LANES, LANES)
        p1_v[sl] = lax.shift_right_logical(idx1_v[sl], 2)
        p2_v[sl] = lax.shift_right_logical(idx2_v[sl], 1)

    iota = lax.iota(jnp.int32, LANES)

    for chunk in range(NCHUNK):
        csl = pl.ds(chunk * CH, CH)
        cps = [
            pltpu.async_copy(er4_hbm.at[p1_v.at[csl]], erq_v, sem),
            pltpu.async_copy(ei4_hbm.at[p1_v.at[csl]], eiq_v, sem),
            pltpu.async_copy(cos2_hbm.at[p2_v.at[csl]], cosp_v, sem),
            pltpu.async_copy(sin2_hbm.at[p2_v.at[csl]], sinp_v, sem),
        ]
        for cp in cps:
            cp.wait()

        for bv in range(CH // LANES):
            gcol = chunk * CH + bv * LANES
            gsl = pl.ds(gcol, LANES)
            row_i = iota + bv * LANES
            par1 = lax.shift_left(idx1_v[gsl] & 3, 6)
            par2 = lax.shift_left(idx2_v[gsl] & 1, 6)

            def fbody(fu, _, row_i=row_i, par1=par1, par2=par2, gsl=gsl):
                for u in range(UNROLL):
                    f = fu * UNROLL + u
                    c1 = par1 + f
                    c2 = par2 + f
                    a = plsc.load_gather(erq_v, [row_i, c1])
                    b = plsc.load_gather(eiq_v, [row_i, c1])
                    c = plsc.load_gather(cosp_v, [row_i, c2])
                    s = plsc.load_gather(sinp_v, [row_i, c2])
                    or_v[f, gsl] = a * c - b * s
                    oi_v[f, gsl] = a * s + b * c
                return _

            lax.fori_loop(0, D // UNROLL, fbody, None)

    pltpu.sync_copy(or_v, outr_hbm.at[:, pl.ds(base, BPW)])
    pltpu.sync_copy(oi_v, outi_hbm.at[:, pl.ds(base, BPW)])


def kernel(e1, r, entity_real, entity_img, relation):
    e1 = e1.astype(jnp.int32)
    r = r.astype(jnp.int32)
    cos_t, sin_t = _cos_sin_tables(relation)
    er4 = entity_real.reshape(250000, 256)
    ei4 = entity_img.reshape(250000, 256)
    cos2 = cos_t.reshape(500, 128)
    sin2 = sin_t.reshape(500, 128)
    outr_t, outi_t = _rotate_sc(e1, r, er4, ei4, cos2, sin2)
    return outr_t.T, outi_t.T


# speedup vs baseline: 1.0503x; 1.0503x over previous
import functools

import jax
import jax.numpy as jnp
from jax import lax
from jax.experimental import pallas as pl
from jax.experimental.pallas import tpu as pltpu
from jax.experimental.pallas import tpu_sc as plsc

B = 16384
D = 64
NC = 2
NS = 16
NW = NC * NS
BPW = B // NW
CH = 64
NCHUNK = BPW // CH
LANES = 16
UNROLL = 4

_SC_PARAMS = pltpu.CompilerParams(needs_layout_passes=False)


def _cs_body(rel_ref, cos_ref, sin_ref):
    cos_ref[...] = jnp.cos(rel_ref[...])
    sin_ref[...] = jnp.sin(rel_ref[...])


def _cos_sin_tables(relation):
    return pl.pallas_call(
        _cs_body,
        out_shape=(
            jax.ShapeDtypeStruct(relation.shape, jnp.float32),
            jax.ShapeDtypeStruct(relation.shape, jnp.float32),
        ),
    )(relation)


def _combine_body(ar_ref, ai_ref, c_ref, s_ref, outr_ref, outi_ref):
    a = ar_ref[...]
    b = ai_ref[...]
    c = c_ref[...]
    s = s_ref[...]
    outr_ref[...] = a * c - b * s
    outi_ref[...] = a * s + b * c


def _combine(ar, ai, c, s):
    grid = 8
    spec = pl.BlockSpec((D, B // grid), lambda i: (0, i))
    return pl.pallas_call(
        _combine_body,
        grid=(grid,),
        in_specs=[spec, spec, spec, spec],
        out_specs=(spec, spec),
        out_shape=(
            jax.ShapeDtypeStruct((D, B), jnp.float32),
            jax.ShapeDtypeStruct((D, B), jnp.float32),
        ),
    )(ar, ai, c, s)


_mesh = plsc.VectorSubcoreMesh(core_axis_name="c", subcore_axis_name="s")


@functools.partial(
    pl.kernel,
    mesh=_mesh,
    compiler_params=_SC_PARAMS,
    out_type=jax.ShapeDtypeStruct((D, B), jnp.float32),
    scratch_types=[
        pltpu.VMEM((BPW,), jnp.int32),
        pltpu.VMEM((BPW,), jnp.int32),
        pltpu.VMEM((CH, 4 * D), jnp.float32),
        pltpu.VMEM((D, BPW), jnp.float32),
        pltpu.SemaphoreType.DMA,
    ],
)
def _gather_entity(idx_hbm, tab4_hbm, out_hbm, idx_v, p_v, q_v, o_v, sem):
    wid = lax.axis_index("s") * NC + lax.axis_index("c")
    base = wid * BPW

    pltpu.sync_copy(idx_hbm.at[pl.ds(base, BPW)], idx_v)
    for v in range(BPW // LANES):
        sl = pl.ds(v * LANES, LANES)
        p_v[sl] = lax.shift_right_logical(idx_v[sl], 2)

    iota = lax.iota(jnp.int32, LANES)

    for chunk in range(NCHUNK):
        csl = pl.ds(chunk * CH, CH)
        pltpu.async_copy(tab4_hbm.at[p_v.at[csl]], q_v, sem).wait()
        for bv in range(CH // LANES):
            gcol = chunk * CH + bv * LANES
            gsl = pl.ds(gcol, LANES)
            row_i = iota + bv * LANES
            par = lax.shift_left(idx_v[gsl] & 3, 6)

            def fbody(fu, _, row_i=row_i, par=par, gsl=gsl):
                for u in range(UNROLL):
                    f = fu * UNROLL + u
                    o_v[f, gsl] = plsc.load_gather(q_v, [row_i, par + f])
                return _

            lax.fori_loop(0, D // UNROLL, fbody, None)

    pltpu.sync_copy(o_v, out_hbm.at[:, pl.ds(base, BPW)])


@functools.partial(
    pl.kernel,
    mesh=_mesh,
    compiler_params=_SC_PARAMS,
    out_type=(
        jax.ShapeDtypeStruct((D, B), jnp.float32),
        jax.ShapeDtypeStruct((D, B), jnp.float32),
    ),
    scratch_types=[
        pltpu.VMEM((BPW,), jnp.int32),
        pltpu.VMEM((BPW,), jnp.int32),
        pltpu.VMEM((CH, 2 * D), jnp.float32),
        pltpu.VMEM((CH, 2 * D), jnp.float32),
        pltpu.VMEM((D, BPW), jnp.float32),
        pltpu.VMEM((D, BPW), jnp.float32),
        pltpu.SemaphoreType.DMA,
    ],
)
def _gather_cs(r_hbm, cos2_hbm, sin2_hbm, outc_hbm, outs_hbm,
               idx_v, p_v, cq_v, sq_v, oc_v, os_v, sem):
    wid = lax.axis_index("s") * NC + lax.axis_index("c")
    base = wid * BPW

    pltpu.sync_copy(r_hbm.at[pl.ds(base, BPW)], idx_v)
    for v in range(BPW // LANES):
        sl = pl.ds(v * LANES, LANES)
        p_v[sl] = lax.shift_right_logical(idx_v[sl], 1)

    iota = lax.iota(jnp.int32, LANES)

    for chunk in range(NCHUNK):
        csl = pl.ds(chunk * CH, CH)
        cps = [
            pltpu.async_copy(cos2_hbm.at[p_v.at[csl]], cq_v, sem),
            pltpu.async_copy(sin2_hbm.at[p_v.at[csl]], sq_v, sem),
        ]
        for cp in cps:
            cp.wait()
        for bv in range(CH // LANES):
            gcol = chunk * CH + bv * LANES
            gsl = pl.ds(gcol, LANES)
            row_i = iota + bv * LANES
            par = lax.shift_left(idx_v[gsl] & 1, 6)

            def fbody(fu, _, row_i=row_i, par=par, gsl=gsl):
                for u in range(UNROLL):
                    f = fu * UNROLL + u
                    oc_v[f, gsl] = plsc.load_gather(cq_v, [row_i, par + f])
                    os_v[f, gsl] = plsc.load_gather(sq_v, [row_i, par + f])
                return _

            lax.fori_loop(0, D // UNROLL, fbody, None)

    pltpu.sync_copy(oc_v, outc_hbm.at[:, pl.ds(base, BPW)])
    pltpu.sync_copy(os_v, outs_hbm.at[:, pl.ds(base, BPW)])


def kernel(e1, r, entity_real, entity_img, relation):
    e1 = e1.astype(jnp.int32)
    r = r.astype(jnp.int32)
    cos_t, sin_t = _cos_sin_tables(relation)
    er4 = entity_real.reshape(250000, 256)
    ei4 = entity_img.reshape(250000, 256)
    ar = _gather_entity(e1, er4)
    ai = _gather_entity(e1, ei4)
    c, s = _gather_cs(r, cos_t.reshape(500, 128), sin_t.reshape(500, 128))
    outr_t, outi_t = _combine(ar, ai, c, s)
    return outr_t.T, outi_t.T
